# Initial kernel scaffold; baseline (speedup 1.0000x reference)
import jax
import jax.numpy as jnp
from jax.experimental import pallas as pl


def kernel(feats, batch_idx, W1, b1, W2, b2):
    raise NotImplementedError("write your pallas kernel here")


# speedup vs baseline: 2.4856x; 2.4856x over previous
import jax
import jax.numpy as jnp
from jax import lax
from jax.experimental import pallas as pl
from jax.experimental.pallas import tpu as pltpu
from jax.experimental.pallas import tpu_sc as plsc

_B = 16
_MAXLEN = 4096
_N = 32768
_DLOC = 16
_DOUT = 256
_HALF = _MAXLEN // 2
_NBLK = _HALF // 16
_FPAD = _N + 2 * _MAXLEN
_FBUF = _HALF + 24


def _sc_body(feats_hbm, bidx_hbm, w1_hbm,
             lfeat_hbm, psums_hbm, counts_hbm,
             bidx_v, feats_v, w1_v, stage_v, acc_v, ps_v, cf_v):
    c = lax.axis_index("c")
    s = lax.axis_index("s")
    w = s * 2 + c
    b = w // 2
    half = w % 2

    pltpu.sync_copy(bidx_hbm, bidx_v)
    pltpu.sync_copy(w1_hbm, w1_v)

    iota = lax.broadcasted_iota(jnp.int32, (16,), 0)

    def lower_bound(target):
        lo = jnp.zeros((16,), jnp.int32)
        for k in range(16):
            stepsz = 32768 >> k
            cand = lo + stepsz
            idx = jnp.minimum(cand, _N) - 1
            vals = plsc.load_gather(bidx_v, [idx])
            pred = (cand <= _N) & (vals < target)
            lo = jnp.where(pred, cand, lo)
        return lo

    starts = lower_bound(iota)
    ends = lower_bound(iota + 1)
    cnts = ends - starts
    cf_v[...] = cnts.astype(jnp.float32)

    zero16i = jnp.zeros((16,), jnp.int32)
    start_b = jnp.sum(jnp.where(iota == b, starts, zero16i))
    cnt_b = jnp.sum(jnp.where(iota == b, cnts, zero16i))
    lb = jnp.minimum(cnt_b, _MAXLEN)
    nvalid = jnp.clip(lb - half * _HALF, 0, _HALF)
    t0 = start_b + half * _HALF
    at0 = (t0 >> 3) << 3
    off = t0 - at0

    fstart = pl.multiple_of(at0 * 4, 32)
    pltpu.sync_copy(feats_hbm.at[pl.ds(fstart, _FBUF * 4)], feats_v)

    zero16 = jnp.zeros((16,), jnp.float32)
    for d in range(_DLOC):
        acc_v[d] = zero16

    w1rows = [w1_v[pl.ds(k * 16, 16)] for k in range(4)]
    w1s = [[jnp.sum(jnp.where(iota == d, w1rows[k], 0.0)) for k in range(4)]
           for d in range(_DLOC)]

    def gather3(j):
        fidx = (off + j * 16 + iota) * 4
        return (plsc.load_gather(feats_v, [fidx]),
                plsc.load_gather(feats_v, [fidx + 1]),
                plsc.load_gather(feats_v, [fidx + 2]))

    def block(j, fs):
        f0, f1, f2 = fs
        fs_next = gather3(j + 1)
        p0 = j * 16
        mask = (p0 + iota) < nvalid
        for d in range(_DLOC):
            v = f0 * w1s[d][0] + f1 * w1s[d][1] + f2 * w1s[d][2] + w1s[d][3]
            v = jnp.maximum(v, 0.0)
            v = jnp.where(mask, v, 0.0)
            stage_v[d, pl.ds(p0, 16)] = v
            acc_v[d] = acc_v[d] + v
        return fs_next

    lax.fori_loop(0, _NBLK, block, gather3(0))

    pltpu.sync_copy(stage_v, lfeat_hbm.at[b, :, pl.ds(half * _HALF, _HALF)])

    psum_row = jnp.zeros((16,), jnp.float32)
    for l in range(16):
        psum_row = psum_row + plsc.load_gather(
            acc_v, [iota, jnp.full((16,), l, jnp.int32)])
    ps_v[...] = psum_row
    pltpu.sync_copy(ps_v, psums_hbm.at[half, b])

    @pl.when(w == 0)
    def _():
        pltpu.sync_copy(cf_v, counts_hbm)


def _tc_body(psums_ref, counts_ref, w2_ref, b2_ref, out_ref):
    pooled_sum = psums_ref[0] + psums_ref[1]
    denom = jnp.maximum(counts_ref[...], 1.0)
    pooled = pooled_sum / denom
    o = jnp.dot(pooled, w2_ref[...], preferred_element_type=jnp.float32)
    o = o + b2_ref[...]
    nrm = jnp.sqrt(jnp.sum(o * o, axis=1, keepdims=True))
    out_ref[...] = o / jnp.maximum(nrm, 1e-12)


def kernel(feats, batch_idx, W1, b1, W2, b2):
    feats_flat = jnp.pad(feats, ((0, _FPAD - _N), (0, 1))).reshape(-1)
    w1b = jnp.concatenate([W1, b1[None, :]], axis=0).reshape(-1)

    mesh = plsc.VectorSubcoreMesh(core_axis_name="c", subcore_axis_name="s")
    sc = pl.kernel(
        _sc_body,
        out_type=(
            jax.ShapeDtypeStruct((_B, _DLOC, _MAXLEN), jnp.float32),
            jax.ShapeDtypeStruct((2, _B, _DLOC), jnp.float32),
            jax.ShapeDtypeStruct((_B,), jnp.float32),
        ),
        mesh=mesh,
        scratch_types=[
            pltpu.VMEM((_N,), jnp.int32),
            pltpu.VMEM((_FBUF * 4,), jnp.float32),
            pltpu.VMEM((64,), jnp.float32),
            pltpu.VMEM((_DLOC, _HALF), jnp.float32),
            pltpu.VMEM((_DLOC, 16), jnp.float32),
            pltpu.VMEM((16,), jnp.float32),
            pltpu.VMEM((16,), jnp.float32),
        ],
        compiler_params=pltpu.CompilerParams(needs_layout_passes=False),
        name="spgap_sc",
    )
    lfeat, psums, counts_f = sc(feats_flat, batch_idx, w1b)

    out = pl.pallas_call(
        _tc_body,
        out_shape=jax.ShapeDtypeStruct((_B, _DOUT), jnp.float32),
    )(psums, counts_f.reshape(_B, 1), W2, b2.reshape(1, _DOUT))
    return out, lfeat


# speedup vs baseline: 3.4704x; 1.3962x over previous
import jax
import jax.numpy as jnp
from jax import lax
from jax.experimental import pallas as pl
from jax.experimental.pallas import tpu as pltpu
from jax.experimental.pallas import tpu_sc as plsc

_B = 16
_MAXLEN = 4096
_N = 32768
_DLOC = 16
_DOUT = 256
_HALF = _MAXLEN // 2
_NBLK = _HALF // 16
_FBUF = _HALF + 32
_AT0MAX = _N - _FBUF


def _sc_body(feats_hbm, bidx_hbm, w1_hbm,
             lfeat_hbm, psums_hbm, counts_hbm,
             bidx_v, feats_v, w1_v, stage_v, acc_v, ps_v, cf_v):
    c = lax.axis_index("c")
    s = lax.axis_index("s")
    w = s * 2 + c
    b = w // 2
    half = w % 2

    pltpu.sync_copy(bidx_hbm, bidx_v)
    pltpu.sync_copy(w1_hbm, w1_v)

    iota = lax.broadcasted_iota(jnp.int32, (16,), 0)

    def lower_bound(target):
        lo = jnp.zeros((16,), jnp.int32)
        for k in range(16):
            stepsz = 32768 >> k
            cand = lo + stepsz
            idx = jnp.minimum(cand, _N) - 1
            vals = plsc.load_gather(bidx_v, [idx])
            pred = (cand <= _N) & (vals < target)
            lo = jnp.where(pred, cand, lo)
        return lo

    starts = lower_bound(iota)
    ends = lower_bound(iota + 1)
    cnts = ends - starts
    cf_v[...] = cnts.astype(jnp.float32)

    zero16i = jnp.zeros((16,), jnp.int32)
    start_b = jnp.sum(jnp.where(iota == b, starts, zero16i))
    cnt_b = jnp.sum(jnp.where(iota == b, cnts, zero16i))
    lb = jnp.minimum(cnt_b, _MAXLEN)
    nvalid = jnp.clip(lb - half * _HALF, 0, _HALF)
    t0 = start_b + half * _HALF
    at0 = jnp.minimum((t0 >> 3) << 3, _AT0MAX)
    off = t0 - at0

    fstart = pl.multiple_of(at0 * 3, 8)
    pltpu.sync_copy(feats_hbm.at[pl.ds(fstart, _FBUF * 3)], feats_v)

    zero16 = jnp.zeros((16,), jnp.float32)

    w1rows = [w1_v[pl.ds(k * 16, 16)] for k in range(4)]
    w1s = [[jnp.sum(jnp.where(iota == d, w1rows[k], 0.0)) for k in range(4)]
           for d in range(_DLOC)]

    def gather3(j):
        rows = jnp.minimum(off + j * 16 + iota, _FBUF - 1)
        fidx = rows * 3
        return (plsc.load_gather(feats_v, [fidx]),
                plsc.load_gather(feats_v, [fidx + 1]),
                plsc.load_gather(feats_v, [fidx + 2]))

    def halfblock(j, fs, accs):
        f0, f1, f2 = fs
        p0 = j * 16
        mask = (p0 + iota) < nvalid
        new_accs = []
        for d in range(_DLOC):
            v = f0 * w1s[d][0] + f1 * w1s[d][1] + f2 * w1s[d][2] + w1s[d][3]
            v = jnp.maximum(v, 0.0)
            v = jnp.where(mask, v, 0.0)
            stage_v[d, pl.ds(p0, 16)] = v
            new_accs.append(accs[d] + v)
        return tuple(new_accs)

    def block(i, carry):
        fs_even, accs = carry
        j0 = i * 2
        fs_odd = gather3(j0 + 1)
        fs_next = gather3(j0 + 2)
        accs = halfblock(j0, fs_even, accs)
        accs = halfblock(j0 + 1, fs_odd, accs)
        return (fs_next, accs)

    _, accs = lax.fori_loop(0, _NBLK // 2, block,
                            (gather3(0), (zero16,) * _DLOC))
    for d in range(_DLOC):
        acc_v[d] = accs[d]

    pltpu.sync_copy(stage_v, lfeat_hbm.at[b, :, pl.ds(half * _HALF, _HALF)])

    psum_row = jnp.zeros((16,), jnp.float32)
    for l in range(16):
        psum_row = psum_row + plsc.load_gather(
            acc_v, [iota, jnp.full((16,), l, jnp.int32)])
    ps_v[...] = psum_row
    pltpu.sync_copy(ps_v, psums_hbm.at[half, b])

    @pl.when(w == 0)
    def _():
        pltpu.sync_copy(cf_v, counts_hbm)


def _tc_body(psums_ref, counts_ref, w2_ref, b2_ref, out_ref):
    pooled_sum = psums_ref[0] + psums_ref[1]
    denom = jnp.maximum(counts_ref[...], 1.0)
    pooled = pooled_sum / denom
    o = jnp.dot(pooled, w2_ref[...], preferred_element_type=jnp.float32)
    o = o + b2_ref[...]
    nrm = jnp.sqrt(jnp.sum(o * o, axis=1, keepdims=True))
    out_ref[...] = o / jnp.maximum(nrm, 1e-12)


def kernel(feats, batch_idx, W1, b1, W2, b2):
    feats_flat = feats.reshape(-1)
    w1b = jnp.concatenate([W1, b1[None, :]], axis=0).reshape(-1)

    mesh = plsc.VectorSubcoreMesh(core_axis_name="c", subcore_axis_name="s")
    sc = pl.kernel(
        _sc_body,
        out_type=(
            jax.ShapeDtypeStruct((_B, _DLOC, _MAXLEN), jnp.float32),
            jax.ShapeDtypeStruct((2, _B, _DLOC), jnp.float32),
            jax.ShapeDtypeStruct((_B,), jnp.float32),
        ),
        mesh=mesh,
        scratch_types=[
            pltpu.VMEM((_N,), jnp.int32),
            pltpu.VMEM((_FBUF * 3,), jnp.float32),
            pltpu.VMEM((64,), jnp.float32),
            pltpu.VMEM((_DLOC, _HALF), jnp.float32),
            pltpu.VMEM((_DLOC, 16), jnp.float32),
            pltpu.VMEM((16,), jnp.float32),
            pltpu.VMEM((16,), jnp.float32),
        ],
        compiler_params=pltpu.CompilerParams(needs_layout_passes=False),
        name="spgap_sc",
    )
    lfeat, psums, counts_f = sc(feats_flat, batch_idx, w1b)

    out = pl.pallas_call(
        _tc_body,
        out_shape=jax.ShapeDtypeStruct((_B, _DOUT), jnp.float32),
    )(psums, counts_f.reshape(_B, 1), W2, b2.reshape(1, _DOUT))
    return out, lfeat


# speedup vs baseline: 5.2174x; 1.5034x over previous
import jax
import jax.numpy as jnp
from jax import lax
from jax.experimental import pallas as pl
from jax.experimental.pallas import tpu as pltpu
from jax.experimental.pallas import tpu_sc as plsc

_B = 16
_MAXLEN = 4096
_N = 32768
_DLOC = 16
_DOUT = 256
_HALF = _MAXLEN // 2
_NBLK = _HALF // 16
_FBUF = _HALF + 256
_AT0MAX = _N - _FBUF


def _sc_body(feats_hbm, bidx_hbm, w1_hbm,
             lfeat_hbm, psums_hbm, counts_hbm,
             bidx_v, feats_v, w1_v, stage_v, acc_v, ps_v, cf_v):
    c = lax.axis_index("c")
    s = lax.axis_index("s")
    w = s * 2 + c
    b = w // 2
    half = w % 2

    pltpu.sync_copy(bidx_hbm, bidx_v)
    pltpu.sync_copy(w1_hbm, w1_v)

    iota = lax.broadcasted_iota(jnp.int32, (16,), 0)

    def lower_bound(target):
        lo = jnp.zeros((16,), jnp.int32)
        for k in range(16):
            stepsz = 32768 >> k
            cand = lo + stepsz
            idx = jnp.minimum(cand, _N) - 1
            vals = plsc.load_gather(bidx_v, [idx])
            pred = (cand <= _N) & (vals < target)
            lo = jnp.where(pred, cand, lo)
        return lo

    starts = lower_bound(iota)
    ends = lower_bound(iota + 1)
    cnts = ends - starts
    cf_v[...] = cnts.astype(jnp.float32)

    zero16i = jnp.zeros((16,), jnp.int32)
    start_b = jnp.sum(jnp.where(iota == b, starts, zero16i))
    cnt_b = jnp.sum(jnp.where(iota == b, cnts, zero16i))
    lb = jnp.minimum(cnt_b, _MAXLEN)
    nvalid = jnp.clip(lb - half * _HALF, 0, _HALF)
    t0 = start_b + half * _HALF
    at0 = jnp.minimum((t0 >> 7) << 7, _AT0MAX)
    off = t0 - at0

    fstart = pl.multiple_of(at0, 128)
    pltpu.sync_copy(feats_hbm.at[:, pl.ds(fstart, _FBUF)], feats_v)

    zero16 = jnp.zeros((16,), jnp.float32)

    w1rows = [w1_v[pl.ds(k * 16, 16)] for k in range(4)]
    w1s = [[jnp.sum(jnp.where(iota == d, w1rows[k], 0.0)) for k in range(4)]
           for d in range(_DLOC)]

    zeros16i = jnp.zeros((16,), jnp.int32)

    def gather3(j):
        rows = jnp.minimum(off + j * 16 + iota, _FBUF - 1)
        return tuple(plsc.load_gather(feats_v, [zeros16i + k, rows])
                     for k in range(3))

    def halfblock(j, fs, accs):
        f0, f1, f2 = fs
        p0 = j * 16
        mask = (p0 + iota) < nvalid
        new_accs = []
        for d in range(_DLOC):
            v = f0 * w1s[d][0] + f1 * w1s[d][1] + f2 * w1s[d][2] + w1s[d][3]
            v = jnp.maximum(v, 0.0)
            v = jnp.where(mask, v, 0.0)
            stage_v[d, pl.ds(p0, 16)] = v
            new_accs.append(accs[d] + v)
        return tuple(new_accs)

    def block(i, carry):
        fs_even, accs = carry
        j0 = i * 2
        fs_odd = gather3(j0 + 1)
        fs_next = gather3(j0 + 2)
        accs = halfblock(j0, fs_even, accs)
        accs = halfblock(j0 + 1, fs_odd, accs)
        return (fs_next, accs)

    _, accs = lax.fori_loop(0, _NBLK // 2, block,
                            (gather3(0), (zero16,) * _DLOC))
    for d in range(_DLOC):
        acc_v[d] = accs[d]

    pltpu.sync_copy(stage_v, lfeat_hbm.at[b, :, pl.ds(half * _HALF, _HALF)])

    psum_row = jnp.zeros((16,), jnp.float32)
    for l in range(16):
        psum_row = psum_row + plsc.load_gather(
            acc_v, [iota, jnp.full((16,), l, jnp.int32)])
    ps_v[...] = psum_row
    pltpu.sync_copy(ps_v, psums_hbm.at[half, b])

    @pl.when(w == 0)
    def _():
        pltpu.sync_copy(cf_v, counts_hbm)


def _tc_body(psums_ref, counts_ref, w2_ref, b2_ref, out_ref):
    pooled_sum = psums_ref[0] + psums_ref[1]
    denom = jnp.maximum(counts_ref[...], 1.0)
    pooled = pooled_sum / denom
    o = jnp.dot(pooled, w2_ref[...], preferred_element_type=jnp.float32)
    o = o + b2_ref[...]
    nrm = jnp.sqrt(jnp.sum(o * o, axis=1, keepdims=True))
    out_ref[...] = o / jnp.maximum(nrm, 1e-12)


def kernel(feats, batch_idx, W1, b1, W2, b2):
    feats_t = feats.T
    w1b = jnp.concatenate([W1, b1[None, :]], axis=0).reshape(-1)

    mesh = plsc.VectorSubcoreMesh(core_axis_name="c", subcore_axis_name="s")
    sc = pl.kernel(
        _sc_body,
        out_type=(
            jax.ShapeDtypeStruct((_B, _DLOC, _MAXLEN), jnp.float32),
            jax.ShapeDtypeStruct((2, _B, _DLOC), jnp.float32),
            jax.ShapeDtypeStruct((_B,), jnp.float32),
        ),
        mesh=mesh,
        scratch_types=[
            pltpu.VMEM((_N,), jnp.int32),
            pltpu.VMEM((3, _FBUF), jnp.float32),
            pltpu.VMEM((64,), jnp.float32),
            pltpu.VMEM((_DLOC, _HALF), jnp.float32),
            pltpu.VMEM((_DLOC, 16), jnp.float32),
            pltpu.VMEM((16,), jnp.float32),
            pltpu.VMEM((16,), jnp.float32),
        ],
        compiler_params=pltpu.CompilerParams(needs_layout_passes=False),
        name="spgap_sc",
    )
    lfeat, psums, counts_f = sc(feats_t, batch_idx, w1b)

    out = pl.pallas_call(
        _tc_body,
        out_shape=jax.ShapeDtypeStruct((_B, _DOUT), jnp.float32),
    )(psums, counts_f.reshape(_B, 1), W2, b2.reshape(1, _DOUT))
    return out, lfeat


# speedup vs baseline: 5.9727x; 1.1448x over previous
import jax
import jax.numpy as jnp
from jax import lax
from jax.experimental import pallas as pl
from jax.experimental.pallas import tpu as pltpu
from jax.experimental.pallas import tpu_sc as plsc

_B = 16
_MAXLEN = 4096
_N = 32768
_DLOC = 16
_DOUT = 256
_HALF = _MAXLEN // 2
_NBLK = _HALF // 16
_FBUF = _MAXLEN + 256
_AT0MAX = _N - _FBUF


def _sc_body(feats_hbm, bidx_hbm, w1_hbm,
             lfeat_hbm, psums_hbm, counts_hbm,
             bidx_v, feats_v, w1_v, stage_v, acc_v, ps_v, cf_v):
    c = lax.axis_index("c")
    s = lax.axis_index("s")
    w = s * 2 + c
    b = w // 2
    dhalf = w % 2

    pltpu.sync_copy(bidx_hbm, bidx_v)
    pltpu.sync_copy(w1_hbm, w1_v)

    iota = lax.broadcasted_iota(jnp.int32, (16,), 0)

    def lower_bound(target):
        lo = jnp.zeros((16,), jnp.int32)
        for k in range(16):
            stepsz = 32768 >> k
            cand = lo + stepsz
            idx = jnp.minimum(cand, _N) - 1
            vals = plsc.load_gather(bidx_v, [idx])
            pred = (cand <= _N) & (vals < target)
            lo = jnp.where(pred, cand, lo)
        return lo

    starts = lower_bound(iota)
    ends = lower_bound(iota + 1)
    cnts = ends - starts
    cf_v[...] = cnts.astype(jnp.float32)

    zero16i = jnp.zeros((16,), jnp.int32)
    start_b = jnp.sum(jnp.where(iota == b, starts, zero16i))
    cnt_b = jnp.sum(jnp.where(iota == b, cnts, zero16i))
    nvalid = jnp.minimum(cnt_b, _MAXLEN)
    at0 = jnp.minimum((start_b >> 7) << 7, _AT0MAX)
    off = start_b - at0

    fstart = pl.multiple_of(at0, 128)
    pltpu.sync_copy(feats_hbm.at[:, pl.ds(fstart, _FBUF)], feats_v)

    zero16 = jnp.zeros((16,), jnp.float32)

    d0 = dhalf * 8
    w1rows = [w1_v[pl.ds(k * 16, 16)] for k in range(4)]
    w1s = [[jnp.sum(jnp.where(iota == d0 + i, w1rows[k], 0.0))
            for k in range(4)] for i in range(8)]

    def gather3(j):
        rows = jnp.minimum(off + j * 16 + iota, _FBUF - 1)
        return tuple(plsc.load_gather(feats_v, [zero16i + k, rows])
                     for k in range(3))

    def halfblock(j, fs, accs):
        f0, f1, f2 = fs
        p0 = j * 16
        mask = (p0 + iota) < nvalid
        new_accs = []
        for i in range(8):
            v = f0 * w1s[i][0] + f1 * w1s[i][1] + f2 * w1s[i][2] + w1s[i][3]
            v = jnp.maximum(v, 0.0)
            v = jnp.where(mask, v, 0.0)
            stage_v[i, pl.ds(p0, 16)] = v
            new_accs.append(accs[i] + v)
        return tuple(new_accs)

    def block(i, carry):
        fs_even, accs = carry
        j0 = i * 2
        fs_odd = gather3(j0 + 1)
        fs_next = gather3(j0 + 2)
        accs = halfblock(j0, fs_even, accs)
        accs = halfblock(j0 + 1, fs_odd, accs)
        return (fs_next, accs)

    ntrip = (nvalid + 31) // 32
    _, accs = lax.fori_loop(0, ntrip, block,
                            (gather3(0), (zero16,) * 8))

    def zblock(j, carry):
        p0 = j * 16
        for i in range(8):
            stage_v[i, pl.ds(p0, 16)] = zero16
        return carry

    lax.fori_loop(ntrip * 2, _MAXLEN // 16, zblock, 0)

    for i in range(8):
        acc_v[i] = accs[i]

    dsl = pl.multiple_of(d0, 8)
    pltpu.sync_copy(stage_v, lfeat_hbm.at[b, pl.ds(dsl, 8), :])

    rowsel = jnp.clip(iota - d0, 0, 7)
    inhalf = (iota >= d0) & (iota < d0 + 8)
    psum_row = jnp.zeros((16,), jnp.float32)
    for l in range(16):
        psum_row = psum_row + plsc.load_gather(
            acc_v, [rowsel, jnp.full((16,), l, jnp.int32)])
    ps_v[...] = jnp.where(inhalf, psum_row, 0.0)
    pltpu.sync_copy(ps_v, psums_hbm.at[dhalf, b])

    @pl.when(w == 0)
    def _():
        pltpu.sync_copy(cf_v, counts_hbm)


def _tc_body(psums_ref, counts_ref, w2_ref, b2_ref, out_ref):
    pooled_sum = psums_ref[0] + psums_ref[1]
    denom = jnp.maximum(counts_ref[...], 1.0)
    pooled = pooled_sum / denom
    o = jnp.dot(pooled, w2_ref[...], preferred_element_type=jnp.float32)
    o = o + b2_ref[...]
    nrm = jnp.sqrt(jnp.sum(o * o, axis=1, keepdims=True))
    out_ref[...] = o / jnp.maximum(nrm, 1e-12)


def kernel(feats, batch_idx, W1, b1, W2, b2):
    feats_t = feats.T
    w1b = jnp.concatenate([W1, b1[None, :]], axis=0).reshape(-1)

    mesh = plsc.VectorSubcoreMesh(core_axis_name="c", subcore_axis_name="s")
    sc = pl.kernel(
        _sc_body,
        out_type=(
            jax.ShapeDtypeStruct((_B, _DLOC, _MAXLEN), jnp.float32),
            jax.ShapeDtypeStruct((2, _B, _DLOC), jnp.float32),
            jax.ShapeDtypeStruct((_B,), jnp.float32),
        ),
        mesh=mesh,
        scratch_types=[
            pltpu.VMEM((_N,), jnp.int32),
            pltpu.VMEM((3, _FBUF), jnp.float32),
            pltpu.VMEM((64,), jnp.float32),
            pltpu.VMEM((8, _MAXLEN), jnp.float32),
            pltpu.VMEM((8, 16), jnp.float32),
            pltpu.VMEM((16,), jnp.float32),
            pltpu.VMEM((16,), jnp.float32),
        ],
        compiler_params=pltpu.CompilerParams(needs_layout_passes=False),
        name="spgap_sc",
    )
    lfeat, psums, counts_f = sc(feats_t, batch_idx, w1b)

    out = pl.pallas_call(
        _tc_body,
        out_shape=jax.ShapeDtypeStruct((_B, _DOUT), jnp.float32),
    )(psums, counts_f.reshape(_B, 1), W2, b2.reshape(1, _DOUT))
    return out, lfeat
